# baseline (speedup 1.0000x reference)
import functools
import math

import jax
import jax.numpy as jnp
from jax import lax
from jax.experimental import pallas as pl
from jax.experimental.pallas import tpu as pltpu
from jax.experimental.pallas import tpu_sc as plsc

LANES = 16
NCORES = 2
NSUB = 16
NW = NCORES * NSUB
K = 64


def _edge_proj_kernel(lo_ref, hi_ref, we_ref, be_ref, o_ref):
    w = we_ref[...]
    bb = be_ref[...]
    ra = jnp.dot(lo_ref[...], w, preferred_element_type=jnp.float32) + bb
    rb = jnp.dot(hi_ref[...], w, preferred_element_type=jnp.float32) + bb
    a = jax.lax.bitcast_convert_type(
        ra.astype(jnp.bfloat16), jnp.uint16).astype(jnp.uint32)
    b = jax.lax.bitcast_convert_type(
        rb.astype(jnp.bfloat16), jnp.uint16).astype(jnp.uint32)
    o_ref[...] = jax.lax.bitcast_convert_type((b << 16) | a, jnp.int32)


def _edge_proj(edge_attr, We, be):
    e_num, ed = edge_attr.shape
    d = We.shape[1]
    half = e_num // 2
    blk = 1600
    assert half % blk == 0
    nb = half // blk
    return pl.pallas_call(
        _edge_proj_kernel,
        grid=(nb,),
        in_specs=[
            pl.BlockSpec((blk, ed), lambda i: (i, 0)),
            pl.BlockSpec((blk, ed), lambda i: (i + nb, 0)),
            pl.BlockSpec((ed, d), lambda i: (0, 0)),
            pl.BlockSpec((1, d), lambda i: (0, 0)),
        ],
        out_specs=pl.BlockSpec((blk, d), lambda i: (i, 0)),
        out_shape=jax.ShapeDtypeStruct((half, d), jnp.int32),
    )(edge_attr, edge_attr, We, be.reshape(1, d))


def _gin_mlp_kernel(apply_relu, h_ref, p0_ref, p1_ref, w_ref, b_ref, o_ref):
    acc = h_ref[...] + p0_ref[0] + p1_ref[0]
    r = jnp.dot(acc, w_ref[...], preferred_element_type=jnp.float32) + b_ref[...]
    if apply_relu:
        r = jnp.maximum(r, 0.0)
    o_ref[...] = r


def _gin_mlp(h, p, W, b, apply_relu):
    n, d = h.shape
    blk = 1000
    spec = pl.BlockSpec((blk, d), lambda i: (i, 0))
    p0spec = pl.BlockSpec((1, blk, d), lambda i: (0, i, 0))
    p1spec = pl.BlockSpec((1, blk, d), lambda i: (1, i, 0))
    wspec = pl.BlockSpec((d, d), lambda i: (0, 0))
    bspec = pl.BlockSpec((1, d), lambda i: (0, 0))
    return pl.pallas_call(
        functools.partial(_gin_mlp_kernel, apply_relu),
        grid=(n // blk,),
        in_specs=[spec, p0spec, p1spec, wspec, bspec],
        out_specs=spec,
        out_shape=jax.ShapeDtypeStruct((n, d), jnp.float32),
    )(h, p, p, W, b.reshape(1, d))


def _make_sc_layer(n, e_rows2, e_pad, d, agg_rows):
    epw = e_pad // NW
    nchunk = epw // K
    kp = K // 2
    assert nchunk % 6 == 0 and nchunk >= 12
    orows = agg_rows // NSUB
    SB = 2
    SI = 6
    nzfull = agg_rows // K
    nzrem = agg_rows - nzfull * K
    mesh = plsc.VectorSubcoreMesh(core_axis_name="c", subcore_axis_name="s")

    @functools.partial(
        pl.kernel,
        out_type=jax.ShapeDtypeStruct((NCORES, agg_rows, d), jnp.float32),
        mesh=mesh,
        scratch_types=[
            pltpu.VMEM((SB, kp, d), jnp.int32),
            pltpu.VMEM((SB, K, d), jnp.float32),
            pltpu.VMEM((SB, K, d), jnp.float32),
            [pltpu.VMEM((K,), jnp.int32)] * SI,
            [pltpu.VMEM((K,), jnp.int32)] * SI,
            pltpu.VMEM_SHARED((agg_rows, d), jnp.float32),
            [pltpu.SemaphoreType.DMA] * SB,
            [pltpu.SemaphoreType.DMA] * SB,
            [pltpu.SemaphoreType.DMA] * SB,
            [pltpu.SemaphoreType.DMA] * SI,
        ],
    )
    def sc_layer(h_hbm, e_hbm, src_hbm, dst_hbm, out_hbm,
                 e_v, h_v, m_v, srcs, dsts, aggr, esems, hsems, ssems, isems):
        c = lax.axis_index("c")
        s = lax.axis_index("s")

        z = m_v.at[0]

        @pl.loop(0, K)
        def _(r):
            for j in range(d // LANES):
                z[r, pl.ds(j * LANES, LANES)] = jnp.zeros((LANES,),
                                                          jnp.float32)

        for kblk in range((nzfull + NSUB - 1) // NSUB):
            stripe = kblk * NSUB + s
            if (kblk + 1) * NSUB <= nzfull:
                pltpu.sync_copy(z, aggr.at[pl.ds(stripe * K, K)])
            else:
                @pl.when(stripe < nzfull)
                def _():
                    pltpu.sync_copy(z, aggr.at[pl.ds(stripe * K, K)])
        if nzrem:
            @pl.when(s == 0)
            def _():
                pltpu.sync_copy(z.at[pl.ds(0, nzrem)],
                                aggr.at[pl.ds(nzfull * K, nzrem)])
        plsc.subcore_barrier()

        base = (c * NSUB + s) * epw

        def idx_slices(j):
            return (src_hbm.at[pl.ds(base + j * K, K)],
                    dst_hbm.at[pl.ds(base + j * K, K)])

        def issue_idx(j, bi):
            sh, dh = idx_slices(j)
            pltpu.async_copy(sh, srcs[bi], isems[bi])
            pltpu.async_copy(dh, dsts[bi], isems[bi])

        def wait_idx(j, bi):
            sh, dh = idx_slices(j)
            pltpu.make_async_copy(sh, srcs[bi], isems[bi]).wait()
            pltpu.make_async_copy(dh, dsts[bi], isems[bi]).wait()

        def e_slice(j):
            off = pl.multiple_of(
                jnp.minimum(base // 2 + j * kp, e_rows2 - kp), 8)
            return e_hbm.at[pl.ds(off, kp)]

        def issue_streams(j, b, bi):
            pltpu.async_copy(e_slice(j), e_v.at[b], esems[b])
            pltpu.async_copy(h_hbm.at[srcs[bi]], h_v.at[b], hsems[b])

        def body(j, t, sswait, prefetch, idx_prefetch):
            b = t % SB
            bi = t % SI
            pltpu.make_async_copy(e_slice(j), e_v.at[b], esems[b]).wait()
            pltpu.make_async_copy(h_hbm.at[srcs[bi]], h_v.at[b],
                                  hsems[b]).wait()
            if sswait:
                pltpu.make_async_copy(m_v.at[b], aggr.at[dsts[bi]],
                                      ssems[b]).wait()
            eb = e_v.at[b]
            hb = h_v.at[b]
            mb = m_v.at[b]

            hi_mask = jnp.full((LANES,), -65536, jnp.int32)

            @plsc.parallel_loop(0, kp)
            def _(q):
                for jj in range(d // LANES):
                    sl = pl.ds(jj * LANES, LANES)
                    xi = eb[q, sl]
                    lo = jax.lax.bitcast_convert_type(
                        jax.lax.shift_left(xi, 16), jnp.float32)
                    hi = jax.lax.bitcast_convert_type(
                        jax.lax.bitwise_and(xi, hi_mask), jnp.float32)
                    mb[2 * q, sl] = jnp.maximum(hb[2 * q, sl] + lo, 0.0)
                    mb[2 * q + 1, sl] = jnp.maximum(
                        hb[2 * q + 1, sl] + hi, 0.0)

            pltpu.async_copy(mb, aggr.at[dsts[bi]], ssems[b], add=True)
            if prefetch:
                bi2 = (t + 2) % SI
                wait_idx(j + 2, bi2)
                issue_streams(j + 2, b, bi2)
            if idx_prefetch:
                issue_idx(j + 4, (t + 4) % SI)

        for j in range(4):
            issue_idx(j, j)
        for j in range(2):
            wait_idx(j, j)
            issue_streams(j, j, j)
        body(0, 0, sswait=False, prefetch=True, idx_prefetch=True)
        body(1, 1, sswait=False, prefetch=True, idx_prefetch=True)
        for t in range(2, 6):
            body(t, t, sswait=True, prefetch=True, idx_prefetch=True)

        @pl.loop(6, nchunk - 6, step=6)
        def _(i):
            for t in range(6):
                body(i + t, t, sswait=True, prefetch=True, idx_prefetch=True)

        for t in range(6):
            j = nchunk - 6 + t
            body(j, t, sswait=True, prefetch=(t < 4), idx_prefetch=(t < 2))

        for t in range(4, 6):
            pltpu.make_async_copy(m_v.at[t % SB], aggr.at[dsts[t % SI]],
                                  ssems[t % SB]).wait()
        plsc.subcore_barrier()

        pltpu.sync_copy(aggr.at[pl.ds(s * orows, orows)],
                        out_hbm.at[c, pl.ds(s * orows, orows)])

    return sc_layer


def kernel(x, edge_index, edge_attr, We, be, W, b):
    n, d = x.shape
    e_num = edge_index.shape[1]

    unit = NW * K * 6
    e_pad = ((e_num + unit - 1) // unit) * unit
    npad = e_pad - e_num
    agg_rows = ((n + 8 + 127) // 128) * 128

    half = e_num // 2
    src = jnp.stack([edge_index[0, :half], edge_index[0, half:]], 1).reshape(-1)
    dst = jnp.stack([edge_index[1, :half], edge_index[1, half:]], 1).reshape(-1)
    if npad:
        pad_ar = jnp.arange(npad, dtype=jnp.int32)
        src = jnp.concatenate([src, pad_ar % n])
        dst = jnp.concatenate([dst, n + pad_ar % (agg_rows - n)])

    e = _edge_proj(edge_attr, We, be)
    sc_layer = _make_sc_layer(n, e_num // 2, e_pad, d, agg_rows)

    h = x
    num_layers = 2
    for i in range(num_layers):
        p = sc_layer(h, e, src, dst)
        h = _gin_mlp(h, p, W, b, apply_relu=(i != num_layers - 1))
    return h


# speedup vs baseline: 1.1260x; 1.1260x over previous
import functools
import math

import jax
import jax.numpy as jnp
from jax import lax
from jax.experimental import pallas as pl
from jax.experimental.pallas import tpu as pltpu
from jax.experimental.pallas import tpu_sc as plsc

LANES = 16
NCORES = 2
NSUB = 16
NW = NCORES * NSUB
K = 56


def _edge_proj_kernel(ea_ref, we_ref, be_ref, o_ref):
    o_ref[...] = (
        jnp.dot(ea_ref[...], we_ref[...], preferred_element_type=jnp.float32)
        + be_ref[...]
    )


def _edge_proj(edge_attr, We, be, e_pad):
    e_num, ed = edge_attr.shape
    d = We.shape[1]
    blk = math.gcd(e_num, e_pad)
    in_blocks = e_num // blk
    return pl.pallas_call(
        _edge_proj_kernel,
        grid=(e_pad // blk,),
        in_specs=[
            pl.BlockSpec((blk, ed), lambda i: (jnp.minimum(i, in_blocks - 1),
                                               0)),
            pl.BlockSpec((ed, d), lambda i: (0, 0)),
            pl.BlockSpec((1, d), lambda i: (0, 0)),
        ],
        out_specs=pl.BlockSpec((blk, d), lambda i: (i, 0)),
        out_shape=jax.ShapeDtypeStruct((e_pad, d), jnp.float32),
    )(edge_attr, We, be.reshape(1, d))


def _gin_mlp_kernel(apply_relu, h_ref, p0_ref, p1_ref, w_ref, b_ref, o_ref):
    acc = h_ref[...] + p0_ref[0] + p1_ref[0]
    r = jnp.dot(acc, w_ref[...], preferred_element_type=jnp.float32) + b_ref[...]
    if apply_relu:
        r = jnp.maximum(r, 0.0)
    o_ref[...] = r


def _gin_mlp(h, p, W, b, apply_relu):
    n, d = h.shape
    blk = 1000
    spec = pl.BlockSpec((blk, d), lambda i: (i, 0))
    p0spec = pl.BlockSpec((1, blk, d), lambda i: (0, i, 0))
    p1spec = pl.BlockSpec((1, blk, d), lambda i: (1, i, 0))
    wspec = pl.BlockSpec((d, d), lambda i: (0, 0))
    bspec = pl.BlockSpec((1, d), lambda i: (0, 0))
    return pl.pallas_call(
        functools.partial(_gin_mlp_kernel, apply_relu),
        grid=(n // blk,),
        in_specs=[spec, p0spec, p1spec, wspec, bspec],
        out_specs=spec,
        out_shape=jax.ShapeDtypeStruct((n, d), jnp.float32),
    )(h, p, p, W, b.reshape(1, d))


def _make_sc_layer(n, e_pad, d, agg_rows):
    epw = e_pad // NW
    nchunk = epw // K
    assert nchunk % 6 == 0 and nchunk >= 12
    orows = agg_rows // NSUB
    SB = 2
    SI = 6
    nzfull = agg_rows // K
    nzrem = agg_rows - nzfull * K
    mesh = plsc.VectorSubcoreMesh(core_axis_name="c", subcore_axis_name="s")

    @functools.partial(
        pl.kernel,
        out_type=jax.ShapeDtypeStruct((NCORES, agg_rows, d), jnp.float32),
        mesh=mesh,
        scratch_types=[
            pltpu.VMEM((SB, K, d), jnp.float32),
            pltpu.VMEM((SB, K, d), jnp.float32),
            pltpu.VMEM((SB, K, d), jnp.float32),
            [pltpu.VMEM((K,), jnp.int32)] * SI,
            [pltpu.VMEM((K,), jnp.int32)] * SI,
            pltpu.VMEM_SHARED((agg_rows, d), jnp.float32),
            [pltpu.SemaphoreType.DMA] * SB,
            [pltpu.SemaphoreType.DMA] * SB,
            [pltpu.SemaphoreType.DMA] * SB,
            [pltpu.SemaphoreType.DMA] * SI,
        ],
    )
    def sc_layer(h_hbm, e_hbm, src_hbm, dst_hbm, out_hbm,
                 e_v, h_v, m_v, srcs, dsts, aggr, esems, hsems, ssems, isems):
        c = lax.axis_index("c")
        s = lax.axis_index("s")

        z = m_v.at[0]

        @pl.loop(0, K)
        def _(r):
            for j in range(d // LANES):
                z[r, pl.ds(j * LANES, LANES)] = jnp.zeros((LANES,),
                                                          jnp.float32)

        for kblk in range((nzfull + NSUB - 1) // NSUB):
            stripe = kblk * NSUB + s
            if (kblk + 1) * NSUB <= nzfull:
                pltpu.sync_copy(z, aggr.at[pl.ds(stripe * K, K)])
            else:
                @pl.when(stripe < nzfull)
                def _():
                    pltpu.sync_copy(z, aggr.at[pl.ds(stripe * K, K)])
        if nzrem:
            @pl.when(s == 0)
            def _():
                pltpu.sync_copy(z.at[pl.ds(0, nzrem)],
                                aggr.at[pl.ds(nzfull * K, nzrem)])
        plsc.subcore_barrier()

        base = (c * NSUB + s) * epw

        def idx_slices(j):
            return (src_hbm.at[pl.ds(base + j * K, K)],
                    dst_hbm.at[pl.ds(base + j * K, K)])

        def issue_idx(j, bi):
            sh, dh = idx_slices(j)
            pltpu.async_copy(sh, srcs[bi], isems[bi])
            pltpu.async_copy(dh, dsts[bi], isems[bi])

        def wait_idx(j, bi):
            sh, dh = idx_slices(j)
            pltpu.make_async_copy(sh, srcs[bi], isems[bi]).wait()
            pltpu.make_async_copy(dh, dsts[bi], isems[bi]).wait()

        def e_slice(j):
            return e_hbm.at[pl.ds(base + j * K, K)]

        def issue_streams(j, b, bi):
            pltpu.async_copy(e_slice(j), e_v.at[b], esems[b])
            pltpu.async_copy(h_hbm.at[srcs[bi]], h_v.at[b], hsems[b])

        def body(j, t, sswait, prefetch, idx_prefetch):
            b = t % SB
            bi = t % SI
            pltpu.make_async_copy(e_slice(j), e_v.at[b], esems[b]).wait()
            pltpu.make_async_copy(h_hbm.at[srcs[bi]], h_v.at[b],
                                  hsems[b]).wait()
            if sswait:
                pltpu.make_async_copy(m_v.at[b], aggr.at[dsts[bi]],
                                      ssems[b]).wait()
            eb = e_v.at[b]
            hb = h_v.at[b]
            mb = m_v.at[b]

            @plsc.parallel_loop(0, K, step=2, unroll=2)
            def _(r):
                for rr in range(2):
                    for jj in range(d // LANES):
                        sl = pl.ds(jj * LANES, LANES)
                        mb[r + rr, sl] = jnp.maximum(
                            eb[r + rr, sl] + hb[r + rr, sl], 0.0)

            pltpu.async_copy(mb, aggr.at[dsts[bi]], ssems[b], add=True)
            if prefetch:
                bi2 = (t + 2) % SI
                wait_idx(j + 2, bi2)
                issue_streams(j + 2, b, bi2)
            if idx_prefetch:
                issue_idx(j + 4, (t + 4) % SI)

        for j in range(4):
            issue_idx(j, j)
        for j in range(2):
            wait_idx(j, j)
            issue_streams(j, j, j)
        body(0, 0, sswait=False, prefetch=True, idx_prefetch=True)
        body(1, 1, sswait=False, prefetch=True, idx_prefetch=True)
        for t in range(2, 6):
            body(t, t, sswait=True, prefetch=True, idx_prefetch=True)

        @pl.loop(6, nchunk - 6, step=6)
        def _(i):
            for t in range(6):
                body(i + t, t, sswait=True, prefetch=True, idx_prefetch=True)

        for t in range(6):
            j = nchunk - 6 + t
            body(j, t, sswait=True, prefetch=(t < 4), idx_prefetch=(t < 2))

        for t in range(4, 6):
            pltpu.make_async_copy(m_v.at[t % SB], aggr.at[dsts[t % SI]],
                                  ssems[t % SB]).wait()
        plsc.subcore_barrier()

        pltpu.sync_copy(aggr.at[pl.ds(s * orows, orows)],
                        out_hbm.at[c, pl.ds(s * orows, orows)])

    return sc_layer


def kernel(x, edge_index, edge_attr, We, be, W, b):
    n, d = x.shape
    e_num = edge_index.shape[1]

    unit = NW * K * 6
    e_pad = ((e_num + unit - 1) // unit) * unit
    npad = e_pad - e_num
    agg_rows = ((n + 8 + 127) // 128) * 128

    src = edge_index[0]
    dst = edge_index[1]
    if npad:
        pad_ar = jnp.arange(npad, dtype=jnp.int32)
        src = jnp.concatenate([src, pad_ar % n])
        dst = jnp.concatenate([dst, n + pad_ar % (agg_rows - n)])

    e = _edge_proj(edge_attr, We, be, e_pad)
    sc_layer = _make_sc_layer(n, e_pad, d, agg_rows)

    h = x
    num_layers = 2
    for i in range(num_layers):
        p = sc_layer(h, e, src, dst)
        h = _gin_mlp(h, p, W, b, apply_relu=(i != num_layers - 1))
    return h
